# Initial kernel scaffold; baseline (speedup 1.0000x reference)
import jax
import jax.numpy as jnp
from jax.experimental import pallas as pl


def kernel(h, edge_index, r, W_fc, b_fc):
    raise NotImplementedError("write your pallas kernel here")


# speedup vs baseline: 42.2761x; 42.2761x over previous
import functools

import jax
import jax.numpy as jnp
from jax import lax
from jax.experimental import pallas as pl
from jax.experimental.pallas import tpu as pltpu
from jax.experimental.pallas import tpu_sc as plsc

_B = 50
_NPG = 200
_N = _B * _NPG
_EPG = 6400
_D = 128
_ADIM = 25
_NW = 32
_GPW = 2


def _build_adj_sc(src2d, dst2d):
    mesh = plsc.VectorSubcoreMesh(core_axis_name="c", subcore_axis_name="s")

    @functools.partial(
        pl.kernel,
        out_type=jax.ShapeDtypeStruct((_B, _NPG * _NPG), jnp.float32),
        mesh=mesh,
        scratch_types=[
            pltpu.VMEM((_EPG,), jnp.int32),
            pltpu.VMEM((_EPG,), jnp.int32),
            pltpu.VMEM((_NPG * _NPG,), jnp.float32),
        ],
        compiler_params=pltpu.CompilerParams(needs_layout_passes=False),
    )
    def sc_kernel(src_hbm, dst_hbm, out_hbm, src_v, dst_v, acc_v):
        wid = lax.axis_index("s") * 2 + lax.axis_index("c")
        zero16 = jnp.zeros((16,), jnp.float32)
        for t in range(_GPW):
            g = wid + t * _NW

            @pl.when(g < _B)
            def _():
                pltpu.sync_copy(src_hbm.at[g], src_v)
                pltpu.sync_copy(dst_hbm.at[g], dst_v)

                @plsc.parallel_loop(0, _NPG * _NPG // 16, unroll=8)
                def _(i):
                    acc_v[pl.ds(i * 16, 16)] = zero16

                base = g * _NPG * (_NPG + 1)

                def body(i, carry):
                    s16 = src_v[pl.ds(i * 16, 16)]
                    d16 = dst_v[pl.ds(i * 16, 16)]
                    idx = s16 * _NPG + d16 - base
                    cnt, last = plsc.scan_count(idx)
                    plsc.addupdate_scatter(
                        acc_v, [idx], cnt.astype(jnp.float32), mask=last
                    )
                    return carry

                lax.fori_loop(0, _EPG // 16, body, 0)
                pltpu.sync_copy(acc_v, out_hbm.at[g])

    return sc_kernel(src2d, dst2d)


def _tc_body(a_ref, h_ref, w_ref, b_ref, adj_ref, hb_ref):
    g = pl.program_id(0)
    a = a_ref[...]
    h = h_ref[...]
    w = w_ref[...]
    b = b_ref[...]

    hw = jax.lax.dot_general(
        h, w, (((1,), (0,)), ((), ())), preferred_element_type=jnp.float32
    )
    hw_ext = jnp.concatenate(
        [hw, jnp.ones((_NPG, 1), jnp.float32)], axis=1
    )
    agg_ext = jax.lax.dot_general(
        a, hw_ext, (((0,), (0,)), ((), ())), preferred_element_type=jnp.float32
    )
    aggw = agg_ext[:, :_ADIM]
    deg = agg_ext[:, _ADIM:]
    logits = aggw / jnp.maximum(deg, 1.0) + b
    m = jnp.max(logits, axis=1, keepdims=True)
    e = jnp.exp(logits - m)
    assign = e / jnp.sum(e, axis=1, keepdims=True)

    h_pool = jax.lax.dot_general(
        assign, h, (((0,), (0,)), ((), ())), preferred_element_type=jnp.float32
    )
    hb_ref[0] = h_pool

    as_ = jax.lax.dot_general(
        a, assign, (((1,), (0,)), ((), ())), preferred_element_type=jnp.float32
    )
    adj_small = jax.lax.dot_general(
        assign, as_, (((0,), (0,)), ((), ())), preferred_element_type=jnp.float32
    )

    ik = lax.broadcasted_iota(jnp.int32, (_ADIM, _B * _ADIM), 0)
    jc = lax.broadcasted_iota(jnp.int32, (_ADIM, _B * _ADIM), 1)
    sel = (jc == ik + g * _ADIM).astype(jnp.float32)
    adj_ref[0] = jax.lax.dot_general(
        adj_small, sel, (((1,), (0,)), ((), ())),
        preferred_element_type=jnp.float32,
    )


def _pool_tc(a, h, w_fc, b2d):
    return pl.pallas_call(
        _tc_body,
        grid=(_B,),
        in_specs=[
            pl.BlockSpec((_NPG, _NPG), lambda g: (g, 0)),
            pl.BlockSpec((_NPG, _D), lambda g: (g, 0)),
            pl.BlockSpec((_D, _ADIM), lambda g: (0, 0)),
            pl.BlockSpec((1, _ADIM), lambda g: (0, 0)),
        ],
        out_specs=[
            pl.BlockSpec((1, _ADIM, _B * _ADIM), lambda g: (g, 0, 0)),
            pl.BlockSpec((1, _ADIM, _D), lambda g: (g, 0, 0)),
        ],
        out_shape=[
            jax.ShapeDtypeStruct((_B, _ADIM, _B * _ADIM), jnp.float32),
            jax.ShapeDtypeStruct((_B, _ADIM, _D), jnp.float32),
        ],
        compiler_params=pltpu.CompilerParams(
            dimension_semantics=("arbitrary",),
        ),
    )(a, h, w_fc, b2d)


def kernel(h, edge_index, r, W_fc, b_fc):
    del r
    src = edge_index[0].reshape(_B, _EPG)
    dst = edge_index[1].reshape(_B, _EPG)
    a_flat = _build_adj_sc(src, dst)
    a = a_flat.reshape(_N, _NPG)
    adj_blocks, h_batched = _pool_tc(
        a, h, W_fc, b_fc.reshape(1, _ADIM)
    )
    adj_new = adj_blocks.reshape(_B * _ADIM, _B * _ADIM)
    return (adj_new, h_batched)


# speedup vs baseline: 45.8766x; 1.0852x over previous
import functools

import jax
import jax.numpy as jnp
from jax import lax
from jax.experimental import pallas as pl
from jax.experimental.pallas import tpu as pltpu
from jax.experimental.pallas import tpu_sc as plsc

_B = 50
_NPG = 200
_N = _B * _NPG
_EPG = 6400
_D = 128
_ADIM = 25
_NW = 32
_GPW = 2


def _build_adj_sc(src2d, dst2d):
    mesh = plsc.VectorSubcoreMesh(core_axis_name="c", subcore_axis_name="s")

    @functools.partial(
        pl.kernel,
        out_type=jax.ShapeDtypeStruct((_B, _NPG * _NPG), jnp.float32),
        mesh=mesh,
        scratch_types=[
            pltpu.VMEM((_EPG,), jnp.int32),
            pltpu.VMEM((_EPG,), jnp.int32),
            pltpu.VMEM((_NPG * _NPG,), jnp.float32),
        ],
        compiler_params=pltpu.CompilerParams(needs_layout_passes=False),
    )
    def sc_kernel(src_hbm, dst_hbm, out_hbm, src_v, dst_v, acc_v):
        wid = lax.axis_index("s") * 2 + lax.axis_index("c")
        zero16 = jnp.zeros((16,), jnp.float32)
        for t in range(_GPW):
            g = wid + t * _NW

            @pl.when(g < _B)
            def _():
                pltpu.sync_copy(src_hbm.at[g], src_v)
                pltpu.sync_copy(dst_hbm.at[g], dst_v)

                @plsc.parallel_loop(0, _NPG * _NPG // 16, unroll=8)
                def _(i):
                    acc_v[pl.ds(i * 16, 16)] = zero16

                base = g * _NPG * (_NPG + 1)

                def body(i, carry):
                    s16 = src_v[pl.ds(i * 16, 16)]
                    d16 = dst_v[pl.ds(i * 16, 16)]
                    idx = s16 * _NPG + d16 - base
                    cnt, last = plsc.scan_count(idx)
                    plsc.addupdate_scatter(
                        acc_v, [idx], cnt.astype(jnp.float32), mask=last
                    )
                    return carry

                lax.fori_loop(0, _EPG // 16, body, 0)
                pltpu.sync_copy(acc_v, out_hbm.at[g])

    return sc_kernel(src2d, dst2d)


_GPB = 10


def _tc_body(a_ref, h_ref, w_ref, b_ref, adj_ref, hb_ref):
    step = pl.program_id(0)
    w = w_ref[...]
    b = b_ref[...]
    for i in range(_GPB):
        a = a_ref[i * _NPG:(i + 1) * _NPG, :]
        h = h_ref[i * _NPG:(i + 1) * _NPG, :]

        hw = jax.lax.dot_general(
            h, w, (((1,), (0,)), ((), ())), preferred_element_type=jnp.float32
        )
        hw_ext = jnp.concatenate(
            [hw, jnp.ones((_NPG, 1), jnp.float32)], axis=1
        )
        agg_ext = jax.lax.dot_general(
            a, hw_ext, (((0,), (0,)), ((), ())),
            preferred_element_type=jnp.float32,
        )
        aggw = agg_ext[:, :_ADIM]
        deg = agg_ext[:, _ADIM:]
        logits = aggw / jnp.maximum(deg, 1.0) + b
        m = jnp.max(logits, axis=1, keepdims=True)
        e = jnp.exp(logits - m)
        assign = e / jnp.sum(e, axis=1, keepdims=True)

        h_pool = jax.lax.dot_general(
            assign, h, (((0,), (0,)), ((), ())),
            preferred_element_type=jnp.float32,
        )
        hb_ref[i] = h_pool

        as_ = jax.lax.dot_general(
            a, assign, (((1,), (0,)), ((), ())),
            preferred_element_type=jnp.float32,
        )
        adj_small = jax.lax.dot_general(
            assign, as_, (((0,), (0,)), ((), ())),
            preferred_element_type=jnp.float32,
        )

        g = step * _GPB + i
        ik = lax.broadcasted_iota(jnp.int32, (_ADIM, _B * _ADIM), 0)
        jc = lax.broadcasted_iota(jnp.int32, (_ADIM, _B * _ADIM), 1)
        sel = (jc == ik + g * _ADIM).astype(jnp.float32)
        adj_ref[i] = jax.lax.dot_general(
            adj_small, sel, (((1,), (0,)), ((), ())),
            preferred_element_type=jnp.float32,
        )


def _pool_tc(a, h, w_fc, b2d):
    return pl.pallas_call(
        _tc_body,
        grid=(_B // _GPB,),
        in_specs=[
            pl.BlockSpec((_GPB * _NPG, _NPG), lambda g: (g, 0)),
            pl.BlockSpec((_GPB * _NPG, _D), lambda g: (g, 0)),
            pl.BlockSpec((_D, _ADIM), lambda g: (0, 0)),
            pl.BlockSpec((1, _ADIM), lambda g: (0, 0)),
        ],
        out_specs=[
            pl.BlockSpec((_GPB, _ADIM, _B * _ADIM), lambda g: (g, 0, 0)),
            pl.BlockSpec((_GPB, _ADIM, _D), lambda g: (g, 0, 0)),
        ],
        out_shape=[
            jax.ShapeDtypeStruct((_B, _ADIM, _B * _ADIM), jnp.float32),
            jax.ShapeDtypeStruct((_B, _ADIM, _D), jnp.float32),
        ],
        compiler_params=pltpu.CompilerParams(
            dimension_semantics=("arbitrary",),
        ),
    )(a, h, w_fc, b2d)


def kernel(h, edge_index, r, W_fc, b_fc):
    del r
    src = edge_index[0].reshape(_B, _EPG)
    dst = edge_index[1].reshape(_B, _EPG)
    a_flat = _build_adj_sc(src, dst)
    a = a_flat.reshape(_N, _NPG)
    adj_blocks, h_batched = _pool_tc(
        a, h, W_fc, b_fc.reshape(1, _ADIM)
    )
    adj_new = adj_blocks.reshape(_B * _ADIM, _B * _ADIM)
    return (adj_new, h_batched)


# speedup vs baseline: 57.1129x; 1.2449x over previous
import functools

import jax
import jax.numpy as jnp
from jax import lax
from jax.experimental import pallas as pl
from jax.experimental.pallas import tpu as pltpu
from jax.experimental.pallas import tpu_sc as plsc

_B = 50
_NPG = 200
_N = _B * _NPG
_EPG = 6400
_E = _B * _EPG
_D = 128
_ADIM = 25
_NW = 32
_GPW = 2


def _build_adj_sc(edge_index, zeros_block):
    mesh = plsc.VectorSubcoreMesh(core_axis_name="c", subcore_axis_name="s")

    @functools.partial(
        pl.kernel,
        out_type=jax.ShapeDtypeStruct((_N, _NPG), jnp.float32),
        mesh=mesh,
        scratch_types=[
            pltpu.VMEM((_EPG,), jnp.int32),
            pltpu.VMEM((_EPG,), jnp.int32),
            pltpu.VMEM((_NPG, _NPG), jnp.float32),
        ],
        compiler_params=pltpu.CompilerParams(needs_layout_passes=False),
    )
    def sc_kernel(edges_hbm, zeros_hbm, out_hbm, src_v, dst_v, acc_v):
        wid = lax.axis_index("s") * 2 + lax.axis_index("c")
        for t in range(_GPW):
            g = wid + t * _NW

            @pl.when(g < _B)
            def _():
                pltpu.sync_copy(edges_hbm.at[0, pl.ds(g * _EPG, _EPG)], src_v)
                pltpu.sync_copy(edges_hbm.at[1, pl.ds(g * _EPG, _EPG)], dst_v)
                pltpu.sync_copy(zeros_hbm, acc_v)
                off = g * _NPG

                def body(i, carry):
                    s16 = src_v[pl.ds(i * 16, 16)]
                    d16 = dst_v[pl.ds(i * 16, 16)]
                    key = s16 * _NPG + d16
                    cnt, last = plsc.scan_count(key)
                    plsc.addupdate_scatter(
                        acc_v,
                        [s16 - off, d16 - off],
                        cnt.astype(jnp.float32),
                        mask=last,
                    )
                    return carry

                lax.fori_loop(0, _EPG // 16, body, 0)
                pltpu.sync_copy(acc_v, out_hbm.at[pl.ds(g * _NPG, _NPG), :])

    return sc_kernel(edge_index, zeros_block)


def _tc_body(a_ref, h_ref, w_ref, b_ref, adj_ref, hb_ref):
    w = w_ref[...]
    b = b_ref[...]
    h_all = h_ref[...]
    hw_all = jax.lax.dot_general(
        h_all, w, (((1,), (0,)), ((), ())), preferred_element_type=jnp.float32
    )
    hwe_all = jnp.concatenate(
        [hw_all, jnp.ones((_N, 1), jnp.float32)], axis=1
    )

    ik = lax.broadcasted_iota(jnp.int32, (_ADIM, _B * _ADIM), 0)
    jc = lax.broadcasted_iota(jnp.int32, (_ADIM, _B * _ADIM), 1)

    for g in range(_B):
        sl = slice(g * _NPG, (g + 1) * _NPG)
        a = a_ref[sl, :]
        h = h_all[sl, :]
        hwe = hwe_all[sl, :]

        agg_ext = jax.lax.dot_general(
            a, hwe, (((0,), (0,)), ((), ())),
            preferred_element_type=jnp.float32,
        )
        aggw = agg_ext[:, :_ADIM]
        deg = agg_ext[:, _ADIM:]
        logits = aggw / jnp.maximum(deg, 1.0) + b
        m = jnp.max(logits, axis=1, keepdims=True)
        e = jnp.exp(logits - m)
        assign = e / jnp.sum(e, axis=1, keepdims=True)

        as_ = jax.lax.dot_general(
            a, assign, (((1,), (0,)), ((), ())),
            preferred_element_type=jnp.float32,
        )
        cat = jnp.concatenate([h, as_], axis=1)
        pooled = jax.lax.dot_general(
            assign, cat, (((0,), (0,)), ((), ())),
            preferred_element_type=jnp.float32,
        )
        hb_ref[g] = pooled[:, :_D]
        adj_small = pooled[:, _D:]

        sel = (jc == ik + g * _ADIM).astype(jnp.float32)
        adj_ref[g * _ADIM:(g + 1) * _ADIM, :] = jax.lax.dot_general(
            adj_small, sel, (((1,), (0,)), ((), ())),
            preferred_element_type=jnp.float32,
        )


def _pool_tc(a, h, w_fc, b2d):
    return pl.pallas_call(
        _tc_body,
        out_shape=[
            jax.ShapeDtypeStruct((_B * _ADIM, _B * _ADIM), jnp.float32),
            jax.ShapeDtypeStruct((_B, _ADIM, _D), jnp.float32),
        ],
    )(a, h, w_fc, b2d)


def kernel(h, edge_index, r, W_fc, b_fc):
    del r
    zeros_block = jnp.zeros((_NPG, _NPG), jnp.float32)
    a = _build_adj_sc(edge_index, zeros_block)
    adj_new, h_batched = _pool_tc(a, h, W_fc, b_fc.reshape(1, _ADIM))
    return (adj_new, h_batched)


# speedup vs baseline: 73.9125x; 1.2941x over previous
import functools

import jax
import jax.numpy as jnp
from jax import lax
from jax.experimental import pallas as pl
from jax.experimental.pallas import tpu as pltpu
from jax.experimental.pallas import tpu_sc as plsc

_B = 50
_NPG = 200
_N = _B * _NPG
_EPG = 6400
_E = _B * _EPG
_D = 128
_ADIM = 25
_NW = 32
_GPW = 2


def _build_adj_sc(edge_index, zeros_block):
    mesh = plsc.VectorSubcoreMesh(core_axis_name="c", subcore_axis_name="s")

    @functools.partial(
        pl.kernel,
        out_type=jax.ShapeDtypeStruct((_N, _NPG), jnp.float32),
        mesh=mesh,
        scratch_types=[
            pltpu.VMEM((_EPG,), jnp.int32),
            pltpu.VMEM((_EPG,), jnp.int32),
            pltpu.VMEM((_NPG, _NPG), jnp.float32),
        ],
        compiler_params=pltpu.CompilerParams(needs_layout_passes=False),
    )
    def sc_kernel(edges_hbm, zeros_hbm, out_hbm, src_v, dst_v, acc_v):
        wid = lax.axis_index("s") * 2 + lax.axis_index("c")
        for t in range(_GPW):
            g = wid + t * _NW

            @pl.when(g < _B)
            def _():
                pltpu.sync_copy(edges_hbm.at[0, pl.ds(g * _EPG, _EPG)], src_v)
                pltpu.sync_copy(edges_hbm.at[1, pl.ds(g * _EPG, _EPG)], dst_v)
                pltpu.sync_copy(zeros_hbm, acc_v)
                off = g * _NPG

                def body(i, carry):
                    s16 = src_v[pl.ds(i * 16, 16)]
                    d16 = dst_v[pl.ds(i * 16, 16)]
                    key = s16 * _NPG + d16
                    cnt, last = plsc.scan_count(key)
                    plsc.addupdate_scatter(
                        acc_v,
                        [s16 - off, d16 - off],
                        cnt.astype(jnp.float32),
                        mask=last,
                    )
                    return carry

                lax.fori_loop(0, _EPG // 16, body, 0)
                pltpu.sync_copy(acc_v, out_hbm.at[pl.ds(g * _NPG, _NPG), :])

    return sc_kernel(edge_index, zeros_block)


def _tc_body(a_ref, h_ref, w_ref, b_ref, adj_ref, hb_ref):
    w = w_ref[...]
    b = b_ref[...]
    h_all = h_ref[...]
    hw_all = jax.lax.dot_general(
        h_all, w, (((1,), (0,)), ((), ())), preferred_element_type=jnp.float32
    )
    hwe_all = jnp.concatenate(
        [hw_all, jnp.ones((_N, 1), jnp.float32)], axis=1
    )

    ik = lax.broadcasted_iota(jnp.int32, (_ADIM, _B * _ADIM), 0)
    jc = lax.broadcasted_iota(jnp.int32, (_ADIM, _B * _ADIM), 1)

    def gsl(g):
        return slice(g * _NPG, (g + 1) * _NPG)

    agg_ext = [
        jax.lax.dot_general(
            a_ref[gsl(g), :], hwe_all[gsl(g), :], (((0,), (0,)), ((), ())),
            preferred_element_type=jnp.float32,
        )
        for g in range(_B)
    ]

    assigns = []
    for g in range(_B):
        aggw = agg_ext[g][:, :_ADIM]
        deg = agg_ext[g][:, _ADIM:]
        logits = aggw / jnp.maximum(deg, 1.0) + b
        m = jnp.max(logits, axis=1, keepdims=True)
        e = jnp.exp(logits - m)
        assigns.append(e / jnp.sum(e, axis=1, keepdims=True))

    as_ = [
        jax.lax.dot_general(
            a_ref[gsl(g), :], assigns[g], (((1,), (0,)), ((), ())),
            preferred_element_type=jnp.float32,
        )
        for g in range(_B)
    ]

    pooled = [
        jax.lax.dot_general(
            assigns[g],
            jnp.concatenate([h_all[gsl(g), :], as_[g]], axis=1),
            (((0,), (0,)), ((), ())),
            preferred_element_type=jnp.float32,
        )
        for g in range(_B)
    ]

    for g in range(_B):
        hb_ref[g] = pooled[g][:, :_D]
        sel = (jc == ik + g * _ADIM).astype(jnp.float32)
        adj_ref[g * _ADIM:(g + 1) * _ADIM, :] = jax.lax.dot_general(
            pooled[g][:, _D:], sel, (((1,), (0,)), ((), ())),
            preferred_element_type=jnp.float32,
        )


def _pool_tc(a, h, w_fc, b2d):
    return pl.pallas_call(
        _tc_body,
        out_shape=[
            jax.ShapeDtypeStruct((_B * _ADIM, _B * _ADIM), jnp.float32),
            jax.ShapeDtypeStruct((_B, _ADIM, _D), jnp.float32),
        ],
    )(a, h, w_fc, b2d)


def kernel(h, edge_index, r, W_fc, b_fc):
    del r
    zeros_block = jnp.zeros((_NPG, _NPG), jnp.float32)
    a = _build_adj_sc(edge_index, zeros_block)
    adj_new, h_batched = _pool_tc(a, h, W_fc, b_fc.reshape(1, _ADIM))
    return (adj_new, h_batched)


# speedup vs baseline: 78.3992x; 1.0607x over previous
import functools

import jax
import jax.numpy as jnp
from jax import lax
from jax.experimental import pallas as pl
from jax.experimental.pallas import tpu as pltpu
from jax.experimental.pallas import tpu_sc as plsc

_B = 50
_NPG = 200
_N = _B * _NPG
_EPG = 6400
_E = _B * _EPG
_D = 128
_ADIM = 25
_NW = 32
_GPW = 2


def _histogram(src_v, dst_v, acc_v, off):
    def body(i, carry):
        for j in range(4):
            s16 = src_v[pl.ds(i * 64 + j * 16, 16)]
            d16 = dst_v[pl.ds(i * 64 + j * 16, 16)]
            key = s16 * _NPG + d16
            cnt, last = plsc.scan_count(key)
            plsc.addupdate_scatter(
                acc_v,
                [s16 - off, d16 - off],
                cnt.astype(jnp.float32),
                mask=last,
            )
        return carry

    lax.fori_loop(0, _EPG // 64, body, 0)


def _build_adj_sc(edge_index, zeros_block):
    mesh = plsc.VectorSubcoreMesh(core_axis_name="c", subcore_axis_name="s")

    @functools.partial(
        pl.kernel,
        out_type=jax.ShapeDtypeStruct((_N, _NPG), jnp.float32),
        mesh=mesh,
        scratch_types=[
            pltpu.VMEM((_EPG,), jnp.int32),
            pltpu.VMEM((_EPG,), jnp.int32),
            pltpu.VMEM((_EPG,), jnp.int32),
            pltpu.VMEM((_EPG,), jnp.int32),
            pltpu.VMEM((_NPG, _NPG), jnp.float32),
            pltpu.VMEM((_NPG, _NPG), jnp.float32),
            pltpu.SemaphoreType.DMA,
            pltpu.SemaphoreType.DMA,
            pltpu.SemaphoreType.DMA,
        ],
        compiler_params=pltpu.CompilerParams(needs_layout_passes=False),
    )
    def sc_kernel(edges_hbm, zeros_hbm, out_hbm, s0, d0, s1, d1, acc0, acc1,
                  sem0, sem1, semo):
        wid = lax.axis_index("s") * 2 + lax.axis_index("c")
        g0 = wid
        g1 = wid + _NW

        c0s = pltpu.async_copy(edges_hbm.at[0, pl.ds(g0 * _EPG, _EPG)], s0, sem0)
        c0d = pltpu.async_copy(edges_hbm.at[1, pl.ds(g0 * _EPG, _EPG)], d0, sem0)
        c0z = pltpu.async_copy(zeros_hbm, acc0, sem0)

        @pl.when(g1 < _B)
        def _():
            pltpu.async_copy(edges_hbm.at[0, pl.ds(g1 * _EPG, _EPG)], s1, sem1)
            pltpu.async_copy(edges_hbm.at[1, pl.ds(g1 * _EPG, _EPG)], d1, sem1)
            pltpu.async_copy(zeros_hbm, acc1, sem1)

        c0s.wait()
        c0d.wait()
        c0z.wait()
        _histogram(s0, d0, acc0, g0 * _NPG)
        co0 = pltpu.async_copy(
            acc0, out_hbm.at[pl.ds(g0 * _NPG, _NPG), :], semo
        )

        @pl.when(g1 < _B)
        def _():
            pltpu.make_async_copy(
                edges_hbm.at[0, pl.ds(g1 * _EPG, _EPG)], s1, sem1
            ).wait()
            pltpu.make_async_copy(
                edges_hbm.at[1, pl.ds(g1 * _EPG, _EPG)], d1, sem1
            ).wait()
            pltpu.make_async_copy(zeros_hbm, acc1, sem1).wait()
            _histogram(s1, d1, acc1, g1 * _NPG)
            pltpu.sync_copy(acc1, out_hbm.at[pl.ds(g1 * _NPG, _NPG), :])

        co0.wait()

    return sc_kernel(edge_index, zeros_block)


def _tc_body(a_ref, h_ref, w_ref, b_ref, adj_ref, hb_ref):
    w = w_ref[...]
    b = b_ref[...]
    h_all = h_ref[...]
    hw_all = jax.lax.dot_general(
        h_all, w, (((1,), (0,)), ((), ())), preferred_element_type=jnp.float32
    )
    hwe_all = jnp.concatenate(
        [hw_all, jnp.ones((_N, 1), jnp.float32)], axis=1
    )

    ik = lax.broadcasted_iota(jnp.int32, (_ADIM, _B * _ADIM), 0)
    jc = lax.broadcasted_iota(jnp.int32, (_ADIM, _B * _ADIM), 1)

    def gsl(g):
        return slice(g * _NPG, (g + 1) * _NPG)

    agg_ext = [
        jax.lax.dot_general(
            a_ref[gsl(g), :], hwe_all[gsl(g), :], (((0,), (0,)), ((), ())),
            preferred_element_type=jnp.float32,
        )
        for g in range(_B)
    ]

    assigns = []
    for g in range(_B):
        aggw = agg_ext[g][:, :_ADIM]
        deg = agg_ext[g][:, _ADIM:]
        logits = aggw / jnp.maximum(deg, 1.0) + b
        m = jnp.max(logits, axis=1, keepdims=True)
        e = jnp.exp(logits - m)
        assigns.append(e / jnp.sum(e, axis=1, keepdims=True))

    as_ = [
        jax.lax.dot_general(
            a_ref[gsl(g), :], assigns[g], (((1,), (0,)), ((), ())),
            preferred_element_type=jnp.float32,
        )
        for g in range(_B)
    ]

    pooled = [
        jax.lax.dot_general(
            assigns[g],
            jnp.concatenate([h_all[gsl(g), :], as_[g]], axis=1),
            (((0,), (0,)), ((), ())),
            preferred_element_type=jnp.float32,
        )
        for g in range(_B)
    ]

    for g in range(_B):
        hb_ref[g] = pooled[g][:, :_D]
        sel = (jc == ik + g * _ADIM).astype(jnp.float32)
        adj_ref[g * _ADIM:(g + 1) * _ADIM, :] = jax.lax.dot_general(
            pooled[g][:, _D:], sel, (((1,), (0,)), ((), ())),
            preferred_element_type=jnp.float32,
        )


def _pool_tc(a, h, w_fc, b2d):
    return pl.pallas_call(
        _tc_body,
        out_shape=[
            jax.ShapeDtypeStruct((_B * _ADIM, _B * _ADIM), jnp.float32),
            jax.ShapeDtypeStruct((_B, _ADIM, _D), jnp.float32),
        ],
    )(a, h, w_fc, b2d)


def kernel(h, edge_index, r, W_fc, b_fc):
    del r
    zeros_block = jnp.zeros((_NPG, _NPG), jnp.float32)
    a = _build_adj_sc(edge_index, zeros_block)
    adj_new, h_batched = _pool_tc(a, h, W_fc, b_fc.reshape(1, _ADIM))
    return (adj_new, h_batched)


# speedup vs baseline: 90.9490x; 1.1601x over previous
import functools

import jax
import jax.numpy as jnp
from jax import lax
from jax.experimental import pallas as pl
from jax.experimental.pallas import tpu as pltpu
from jax.experimental.pallas import tpu_sc as plsc

_B = 50
_NPG = 200
_N = _B * _NPG
_EPG = 6400
_E = _B * _EPG
_D = 128
_ADIM = 25
_NW = 32
_GPW = 2


def _histogram(src_v, dst_v, acc_v, off):
    @plsc.parallel_loop(0, _EPG // 16, unroll=8)
    def _(i):
        s16 = src_v[pl.ds(i * 16, 16)]
        d16 = dst_v[pl.ds(i * 16, 16)]
        key = s16 * _NPG + d16
        cnt, last = plsc.scan_count(key)
        plsc.addupdate_scatter(
            acc_v,
            [s16 - off, d16 - off],
            cnt.astype(jnp.float32),
            mask=last,
        )


def _build_adj_sc(edge_index, zeros_block):
    mesh = plsc.VectorSubcoreMesh(core_axis_name="c", subcore_axis_name="s")

    @functools.partial(
        pl.kernel,
        out_type=jax.ShapeDtypeStruct((_N, _NPG), jnp.float32),
        mesh=mesh,
        scratch_types=[
            pltpu.VMEM((_EPG,), jnp.int32),
            pltpu.VMEM((_EPG,), jnp.int32),
            pltpu.VMEM((_EPG,), jnp.int32),
            pltpu.VMEM((_EPG,), jnp.int32),
            pltpu.VMEM((_NPG, _NPG), jnp.float32),
            pltpu.VMEM((_NPG, _NPG), jnp.float32),
            pltpu.SemaphoreType.DMA,
            pltpu.SemaphoreType.DMA,
            pltpu.SemaphoreType.DMA,
        ],
        compiler_params=pltpu.CompilerParams(needs_layout_passes=False),
    )
    def sc_kernel(edges_hbm, zeros_hbm, out_hbm, s0, d0, s1, d1, acc0, acc1,
                  sem0, sem1, semo):
        wid = lax.axis_index("s") * 2 + lax.axis_index("c")
        g0 = wid
        g1 = wid + _NW

        c0s = pltpu.async_copy(edges_hbm.at[0, pl.ds(g0 * _EPG, _EPG)], s0, sem0)
        c0d = pltpu.async_copy(edges_hbm.at[1, pl.ds(g0 * _EPG, _EPG)], d0, sem0)
        c0z = pltpu.async_copy(zeros_hbm, acc0, sem0)

        @pl.when(g1 < _B)
        def _():
            pltpu.async_copy(edges_hbm.at[0, pl.ds(g1 * _EPG, _EPG)], s1, sem1)
            pltpu.async_copy(edges_hbm.at[1, pl.ds(g1 * _EPG, _EPG)], d1, sem1)
            pltpu.async_copy(zeros_hbm, acc1, sem1)

        c0s.wait()
        c0d.wait()
        c0z.wait()
        _histogram(s0, d0, acc0, g0 * _NPG)
        co0 = pltpu.async_copy(
            acc0, out_hbm.at[pl.ds(g0 * _NPG, _NPG), :], semo
        )

        @pl.when(g1 < _B)
        def _():
            pltpu.make_async_copy(
                edges_hbm.at[0, pl.ds(g1 * _EPG, _EPG)], s1, sem1
            ).wait()
            pltpu.make_async_copy(
                edges_hbm.at[1, pl.ds(g1 * _EPG, _EPG)], d1, sem1
            ).wait()
            pltpu.make_async_copy(zeros_hbm, acc1, sem1).wait()
            _histogram(s1, d1, acc1, g1 * _NPG)
            pltpu.sync_copy(acc1, out_hbm.at[pl.ds(g1 * _NPG, _NPG), :])

        co0.wait()

    return sc_kernel(edge_index, zeros_block)


_GPS = 10


def _tc_body(a_ref, h_ref, w_ref, b_ref, adj_ref, hb_ref, adjs_scr):
    s = pl.program_id(0)
    w = w_ref[...]
    b = b_ref[...]
    h_all = h_ref[...]
    hw_all = jax.lax.dot_general(
        h_all, w, (((1,), (0,)), ((), ())), preferred_element_type=jnp.float32
    )
    hwe_all = jnp.concatenate(
        [hw_all, jnp.ones((_GPS * _NPG, 1), jnp.float32)], axis=1
    )

    ik = lax.broadcasted_iota(jnp.int32, (_ADIM, _B * _ADIM), 0)
    jc = lax.broadcasted_iota(jnp.int32, (_ADIM, _B * _ADIM), 1)

    def gsl(i):
        return slice(i * _NPG, (i + 1) * _NPG)

    agg_ext = [
        jax.lax.dot_general(
            a_ref[gsl(i), :], hwe_all[gsl(i), :], (((0,), (0,)), ((), ())),
            preferred_element_type=jnp.float32,
        )
        for i in range(_GPS)
    ]

    assigns = []
    for i in range(_GPS):
        aggw = agg_ext[i][:, :_ADIM]
        deg = agg_ext[i][:, _ADIM:]
        logits = aggw / jnp.maximum(deg, 1.0) + b
        m = jnp.max(logits, axis=1, keepdims=True)
        e = jnp.exp(logits - m)
        assigns.append(e / jnp.sum(e, axis=1, keepdims=True))

    as_ = [
        jax.lax.dot_general(
            a_ref[gsl(i), :], assigns[i], (((1,), (0,)), ((), ())),
            preferred_element_type=jnp.float32,
        )
        for i in range(_GPS)
    ]

    pooled = [
        jax.lax.dot_general(
            assigns[i],
            jnp.concatenate([h_all[gsl(i), :], as_[i]], axis=1),
            (((0,), (0,)), ((), ())),
            preferred_element_type=jnp.float32,
        )
        for i in range(_GPS)
    ]

    for i in range(_GPS):
        hb_ref[i] = pooled[i][:, :_D]
        adjs_scr[pl.ds(s * _GPS + i, 1)] = pooled[i][jnp.newaxis, :, _D:]

    @pl.when(s == _B // _GPS - 1)
    def _():
        for g in range(_B):
            sel = (jc == ik + g * _ADIM).astype(jnp.float32)
            adj_ref[g * _ADIM:(g + 1) * _ADIM, :] = jax.lax.dot_general(
                adjs_scr[g], sel, (((1,), (0,)), ((), ())),
                preferred_element_type=jnp.float32,
            )


def _pool_tc(a, h, w_fc, b2d):
    return pl.pallas_call(
        _tc_body,
        grid=(_B // _GPS,),
        in_specs=[
            pl.BlockSpec((_GPS * _NPG, _NPG), lambda s: (s, 0)),
            pl.BlockSpec((_GPS * _NPG, _D), lambda s: (s, 0)),
            pl.BlockSpec((_D, _ADIM), lambda s: (0, 0)),
            pl.BlockSpec((1, _ADIM), lambda s: (0, 0)),
        ],
        out_specs=[
            pl.BlockSpec((_B * _ADIM, _B * _ADIM), lambda s: (0, 0)),
            pl.BlockSpec((_GPS, _ADIM, _D), lambda s: (s, 0, 0)),
        ],
        out_shape=[
            jax.ShapeDtypeStruct((_B * _ADIM, _B * _ADIM), jnp.float32),
            jax.ShapeDtypeStruct((_B, _ADIM, _D), jnp.float32),
        ],
        scratch_shapes=[pltpu.VMEM((_B, _ADIM, _ADIM), jnp.float32)],
        compiler_params=pltpu.CompilerParams(
            dimension_semantics=("arbitrary",),
        ),
    )(a, h, w_fc, b2d)


def kernel(h, edge_index, r, W_fc, b_fc):
    del r
    zeros_block = jnp.zeros((_NPG, _NPG), jnp.float32)
    a = _build_adj_sc(edge_index, zeros_block)
    adj_new, h_batched = _pool_tc(a, h, W_fc, b_fc.reshape(1, _ADIM))
    return (adj_new, h_batched)


# speedup vs baseline: 91.0877x; 1.0015x over previous
import functools

import jax
import jax.numpy as jnp
from jax import lax
from jax.experimental import pallas as pl
from jax.experimental.pallas import tpu as pltpu
from jax.experimental.pallas import tpu_sc as plsc

_B = 50
_NPG = 200
_N = _B * _NPG
_EPG = 6400
_E = _B * _EPG
_D = 128
_ADIM = 25
_NW = 32
_GPW = 2


def _histogram(src_v, dst_v, acc_v, off):
    ones16 = jnp.ones((16,), jnp.float32)

    @plsc.parallel_loop(0, _EPG // 16, unroll=8)
    def _(i):
        s16 = src_v[pl.ds(i * 16, 16)]
        d16 = dst_v[pl.ds(i * 16, 16)]
        plsc.addupdate_scatter(acc_v, [s16 - off, d16 - off], ones16)


def _build_adj_sc(edge_index, zeros_block):
    mesh = plsc.VectorSubcoreMesh(core_axis_name="c", subcore_axis_name="s")

    @functools.partial(
        pl.kernel,
        out_type=jax.ShapeDtypeStruct((_N, _NPG), jnp.float32),
        mesh=mesh,
        scratch_types=[
            pltpu.VMEM((_EPG,), jnp.int32),
            pltpu.VMEM((_EPG,), jnp.int32),
            pltpu.VMEM((_EPG,), jnp.int32),
            pltpu.VMEM((_EPG,), jnp.int32),
            pltpu.VMEM((_NPG, _NPG), jnp.float32),
            pltpu.VMEM((_NPG, _NPG), jnp.float32),
            pltpu.SemaphoreType.DMA,
            pltpu.SemaphoreType.DMA,
            pltpu.SemaphoreType.DMA,
        ],
        compiler_params=pltpu.CompilerParams(needs_layout_passes=False),
    )
    def sc_kernel(edges_hbm, zeros_hbm, out_hbm, s0, d0, s1, d1, acc0, acc1,
                  sem0, sem1, semo):
        wid = lax.axis_index("s") * 2 + lax.axis_index("c")
        g0 = wid
        g1 = wid + _NW

        c0s = pltpu.async_copy(edges_hbm.at[0, pl.ds(g0 * _EPG, _EPG)], s0, sem0)
        c0d = pltpu.async_copy(edges_hbm.at[1, pl.ds(g0 * _EPG, _EPG)], d0, sem0)
        c0z = pltpu.async_copy(zeros_hbm, acc0, sem0)

        @pl.when(g1 < _B)
        def _():
            pltpu.async_copy(edges_hbm.at[0, pl.ds(g1 * _EPG, _EPG)], s1, sem1)
            pltpu.async_copy(edges_hbm.at[1, pl.ds(g1 * _EPG, _EPG)], d1, sem1)
            pltpu.async_copy(zeros_hbm, acc1, sem1)

        c0s.wait()
        c0d.wait()
        c0z.wait()
        _histogram(s0, d0, acc0, g0 * _NPG)
        co0 = pltpu.async_copy(
            acc0, out_hbm.at[pl.ds(g0 * _NPG, _NPG), :], semo
        )

        @pl.when(g1 < _B)
        def _():
            pltpu.make_async_copy(
                edges_hbm.at[0, pl.ds(g1 * _EPG, _EPG)], s1, sem1
            ).wait()
            pltpu.make_async_copy(
                edges_hbm.at[1, pl.ds(g1 * _EPG, _EPG)], d1, sem1
            ).wait()
            pltpu.make_async_copy(zeros_hbm, acc1, sem1).wait()
            _histogram(s1, d1, acc1, g1 * _NPG)
            pltpu.sync_copy(acc1, out_hbm.at[pl.ds(g1 * _NPG, _NPG), :])

        co0.wait()

    return sc_kernel(edge_index, zeros_block)


_GPS = 10


def _tc_body(a_ref, h_ref, w_ref, b_ref, adj_ref, hb_ref, adjs_scr):
    s = pl.program_id(0)
    w = w_ref[...]
    b = b_ref[...]
    h_all = h_ref[...]
    hw_all = jax.lax.dot_general(
        h_all, w, (((1,), (0,)), ((), ())), preferred_element_type=jnp.float32
    )
    hwe_all = jnp.concatenate(
        [hw_all, jnp.ones((_GPS * _NPG, 1), jnp.float32)], axis=1
    )

    ik = lax.broadcasted_iota(jnp.int32, (_ADIM, _B * _ADIM), 0)
    jc = lax.broadcasted_iota(jnp.int32, (_ADIM, _B * _ADIM), 1)

    def gsl(i):
        return slice(i * _NPG, (i + 1) * _NPG)

    agg_ext = [
        jax.lax.dot_general(
            a_ref[gsl(i), :], hwe_all[gsl(i), :], (((0,), (0,)), ((), ())),
            preferred_element_type=jnp.float32,
        )
        for i in range(_GPS)
    ]

    assigns = []
    for i in range(_GPS):
        aggw = agg_ext[i][:, :_ADIM]
        deg = agg_ext[i][:, _ADIM:]
        logits = aggw / jnp.maximum(deg, 1.0) + b
        m = jnp.max(logits, axis=1, keepdims=True)
        e = jnp.exp(logits - m)
        assigns.append(e / jnp.sum(e, axis=1, keepdims=True))

    as_ = [
        jax.lax.dot_general(
            a_ref[gsl(i), :], assigns[i], (((1,), (0,)), ((), ())),
            preferred_element_type=jnp.float32,
        )
        for i in range(_GPS)
    ]

    pooled = [
        jax.lax.dot_general(
            assigns[i],
            jnp.concatenate([h_all[gsl(i), :], as_[i]], axis=1),
            (((0,), (0,)), ((), ())),
            preferred_element_type=jnp.float32,
        )
        for i in range(_GPS)
    ]

    for i in range(_GPS):
        hb_ref[i] = pooled[i][:, :_D]
        adjs_scr[pl.ds(s * _GPS + i, 1)] = pooled[i][jnp.newaxis, :, _D:]

    @pl.when(s == _B // _GPS - 1)
    def _():
        for g in range(_B):
            sel = (jc == ik + g * _ADIM).astype(jnp.float32)
            adj_ref[g * _ADIM:(g + 1) * _ADIM, :] = jax.lax.dot_general(
                adjs_scr[g], sel, (((1,), (0,)), ((), ())),
                preferred_element_type=jnp.float32,
            )


def _pool_tc(a, h, w_fc, b2d):
    return pl.pallas_call(
        _tc_body,
        grid=(_B // _GPS,),
        in_specs=[
            pl.BlockSpec((_GPS * _NPG, _NPG), lambda s: (s, 0)),
            pl.BlockSpec((_GPS * _NPG, _D), lambda s: (s, 0)),
            pl.BlockSpec((_D, _ADIM), lambda s: (0, 0)),
            pl.BlockSpec((1, _ADIM), lambda s: (0, 0)),
        ],
        out_specs=[
            pl.BlockSpec((_B * _ADIM, _B * _ADIM), lambda s: (0, 0)),
            pl.BlockSpec((_GPS, _ADIM, _D), lambda s: (s, 0, 0)),
        ],
        out_shape=[
            jax.ShapeDtypeStruct((_B * _ADIM, _B * _ADIM), jnp.float32),
            jax.ShapeDtypeStruct((_B, _ADIM, _D), jnp.float32),
        ],
        scratch_shapes=[pltpu.VMEM((_B, _ADIM, _ADIM), jnp.float32)],
        compiler_params=pltpu.CompilerParams(
            dimension_semantics=("arbitrary",),
        ),
    )(a, h, w_fc, b2d)


def kernel(h, edge_index, r, W_fc, b_fc):
    del r
    zeros_block = jnp.zeros((_NPG, _NPG), jnp.float32)
    a = _build_adj_sc(edge_index, zeros_block)
    adj_new, h_batched = _pool_tc(a, h, W_fc, b_fc.reshape(1, _ADIM))
    return (adj_new, h_batched)


# speedup vs baseline: 111.3638x; 1.2226x over previous
import functools

import jax
import jax.numpy as jnp
from jax import lax
from jax.experimental import pallas as pl
from jax.experimental.pallas import tpu as pltpu
from jax.experimental.pallas import tpu_sc as plsc

_B = 50
_NPG = 200
_N = _B * _NPG
_EPG = 6400
_E = _B * _EPG
_D = 128
_ADIM = 25
_NW = 32
_GPW = 2


def _histogram(src_v, dst_v, acc_v, off):
    ones16 = jnp.ones((16,), jnp.float32)

    @plsc.parallel_loop(0, _EPG // 16, unroll=8)
    def _(i):
        s16 = src_v[pl.ds(i * 16, 16)]
        d16 = dst_v[pl.ds(i * 16, 16)]
        plsc.addupdate_scatter(acc_v, [s16 - off, d16 - off], ones16)


def _zero_acc(acc_v):
    z = jnp.zeros((16,), jnp.float32)

    @plsc.parallel_loop(0, _NPG, unroll=4)
    def _(r):
        for c in range(_NPG // 16):
            acc_v[r, pl.ds(c * 16, 16)] = z
        acc_v[r, pl.ds(_NPG - 16, 16)] = z


def _build_adj_sc(edge_index):
    mesh = plsc.VectorSubcoreMesh(core_axis_name="c", subcore_axis_name="s")

    @functools.partial(
        pl.kernel,
        out_type=jax.ShapeDtypeStruct((_N, _NPG), jnp.float32),
        mesh=mesh,
        scratch_types=[
            pltpu.VMEM((_EPG,), jnp.int32),
            pltpu.VMEM((_EPG,), jnp.int32),
            pltpu.VMEM((_EPG,), jnp.int32),
            pltpu.VMEM((_EPG,), jnp.int32),
            pltpu.VMEM((_NPG, _NPG), jnp.float32),
            pltpu.VMEM((_NPG, _NPG), jnp.float32),
            pltpu.SemaphoreType.DMA,
            pltpu.SemaphoreType.DMA,
            pltpu.SemaphoreType.DMA,
        ],
        compiler_params=pltpu.CompilerParams(needs_layout_passes=False),
    )
    def sc_kernel(edges_hbm, out_hbm, s0, d0, s1, d1, acc0, acc1,
                  sem0, sem1, semo):
        wid = lax.axis_index("s") * 2 + lax.axis_index("c")
        g0 = wid
        g1 = wid + _NW

        c0s = pltpu.async_copy(edges_hbm.at[0, pl.ds(g0 * _EPG, _EPG)], s0, sem0)
        c0d = pltpu.async_copy(edges_hbm.at[1, pl.ds(g0 * _EPG, _EPG)], d0, sem0)

        @pl.when(g1 < _B)
        def _():
            pltpu.async_copy(edges_hbm.at[0, pl.ds(g1 * _EPG, _EPG)], s1, sem1)
            pltpu.async_copy(edges_hbm.at[1, pl.ds(g1 * _EPG, _EPG)], d1, sem1)

        _zero_acc(acc0)
        c0s.wait()
        c0d.wait()
        _histogram(s0, d0, acc0, g0 * _NPG)
        co0 = pltpu.async_copy(
            acc0, out_hbm.at[pl.ds(g0 * _NPG, _NPG), :], semo
        )

        @pl.when(g1 < _B)
        def _():
            _zero_acc(acc1)
            pltpu.make_async_copy(
                edges_hbm.at[0, pl.ds(g1 * _EPG, _EPG)], s1, sem1
            ).wait()
            pltpu.make_async_copy(
                edges_hbm.at[1, pl.ds(g1 * _EPG, _EPG)], d1, sem1
            ).wait()
            _histogram(s1, d1, acc1, g1 * _NPG)
            pltpu.sync_copy(acc1, out_hbm.at[pl.ds(g1 * _NPG, _NPG), :])

        co0.wait()

    return sc_kernel(edge_index)


_GPS = 10


def _tc_body(a_ref, h_ref, w_ref, b_ref, adj_ref, hb_ref, adjs_scr):
    s = pl.program_id(0)
    w = w_ref[...]
    b = b_ref[...]
    h_all = h_ref[...]
    hw_all = jax.lax.dot_general(
        h_all, w, (((1,), (0,)), ((), ())), preferred_element_type=jnp.float32
    )
    hwe_all = jnp.concatenate(
        [hw_all, jnp.ones((_GPS * _NPG, 1), jnp.float32)], axis=1
    )

    ik = lax.broadcasted_iota(jnp.int32, (_ADIM, _B * _ADIM), 0)
    jc = lax.broadcasted_iota(jnp.int32, (_ADIM, _B * _ADIM), 1)

    def gsl(i):
        return slice(i * _NPG, (i + 1) * _NPG)

    agg_ext = [
        jax.lax.dot_general(
            a_ref[gsl(i), :], hwe_all[gsl(i), :], (((0,), (0,)), ((), ())),
            preferred_element_type=jnp.float32,
        )
        for i in range(_GPS)
    ]

    assigns = []
    for i in range(_GPS):
        aggw = agg_ext[i][:, :_ADIM]
        deg = agg_ext[i][:, _ADIM:]
        logits = aggw / jnp.maximum(deg, 1.0) + b
        m = jnp.max(logits, axis=1, keepdims=True)
        e = jnp.exp(logits - m)
        assigns.append(e / jnp.sum(e, axis=1, keepdims=True))

    as_ = [
        jax.lax.dot_general(
            a_ref[gsl(i), :], assigns[i], (((1,), (0,)), ((), ())),
            preferred_element_type=jnp.float32,
        )
        for i in range(_GPS)
    ]

    pooled = [
        jax.lax.dot_general(
            assigns[i],
            jnp.concatenate([h_all[gsl(i), :], as_[i]], axis=1),
            (((0,), (0,)), ((), ())),
            preferred_element_type=jnp.float32,
        )
        for i in range(_GPS)
    ]

    for i in range(_GPS):
        hb_ref[i] = pooled[i][:, :_D]
        adjs_scr[pl.ds(s * _GPS + i, 1)] = pooled[i][jnp.newaxis, :, _D:]

    @pl.when(s == _B // _GPS - 1)
    def _():
        for g in range(_B):
            sel = (jc == ik + g * _ADIM).astype(jnp.float32)
            adj_ref[g * _ADIM:(g + 1) * _ADIM, :] = jax.lax.dot_general(
                adjs_scr[g], sel, (((1,), (0,)), ((), ())),
                preferred_element_type=jnp.float32,
            )


def _pool_tc(a, h, w_fc, b2d):
    return pl.pallas_call(
        _tc_body,
        grid=(_B // _GPS,),
        in_specs=[
            pl.BlockSpec((_GPS * _NPG, _NPG), lambda s: (s, 0)),
            pl.BlockSpec((_GPS * _NPG, _D), lambda s: (s, 0)),
            pl.BlockSpec((_D, _ADIM), lambda s: (0, 0)),
            pl.BlockSpec((1, _ADIM), lambda s: (0, 0)),
        ],
        out_specs=[
            pl.BlockSpec((_B * _ADIM, _B * _ADIM), lambda s: (0, 0)),
            pl.BlockSpec((_GPS, _ADIM, _D), lambda s: (s, 0, 0)),
        ],
        out_shape=[
            jax.ShapeDtypeStruct((_B * _ADIM, _B * _ADIM), jnp.float32),
            jax.ShapeDtypeStruct((_B, _ADIM, _D), jnp.float32),
        ],
        scratch_shapes=[pltpu.VMEM((_B, _ADIM, _ADIM), jnp.float32)],
        compiler_params=pltpu.CompilerParams(
            dimension_semantics=("arbitrary",),
        ),
    )(a, h, w_fc, b2d)


def kernel(h, edge_index, r, W_fc, b_fc):
    del r
    a = _build_adj_sc(edge_index)
    adj_new, h_batched = _pool_tc(a, h, W_fc, b_fc.reshape(1, _ADIM))
    return (adj_new, h_batched)


# speedup vs baseline: 111.6527x; 1.0026x over previous
import functools

import jax
import jax.numpy as jnp
from jax import lax
from jax.experimental import pallas as pl
from jax.experimental.pallas import tpu as pltpu
from jax.experimental.pallas import tpu_sc as plsc

_B = 50
_NPG = 200
_N = _B * _NPG
_EPG = 6400
_E = _B * _EPG
_D = 128
_ADIM = 25
_NW = 32
_GPW = 2


def _histogram(src_v, dst_v, acc_v, off):
    ones16 = jnp.ones((16,), jnp.float32)

    @plsc.parallel_loop(0, _EPG // 16, unroll=8)
    def _(i):
        s16 = src_v[pl.ds(i * 16, 16)]
        d16 = dst_v[pl.ds(i * 16, 16)]
        plsc.addupdate_scatter(acc_v, [s16 - off, d16 - off], ones16)


def _zero_acc(acc_v):
    z = jnp.zeros((16,), jnp.float32)

    @plsc.parallel_loop(0, _NPG, unroll=4)
    def _(r):
        for c in range(_NPG // 16):
            acc_v[r, pl.ds(c * 16, 16)] = z
        acc_v[r, pl.ds(_NPG - 16, 16)] = z


def _build_adj_sc(edge_index):
    mesh = plsc.VectorSubcoreMesh(core_axis_name="c", subcore_axis_name="s")

    @functools.partial(
        pl.kernel,
        out_type=jax.ShapeDtypeStruct((_N, _NPG), jnp.float32),
        mesh=mesh,
        scratch_types=[
            pltpu.VMEM((_EPG,), jnp.int32),
            pltpu.VMEM((_EPG,), jnp.int32),
            pltpu.VMEM((_EPG,), jnp.int32),
            pltpu.VMEM((_EPG,), jnp.int32),
            pltpu.VMEM((_NPG, _NPG), jnp.float32),
            pltpu.VMEM((_NPG, _NPG), jnp.float32),
            pltpu.SemaphoreType.DMA,
            pltpu.SemaphoreType.DMA,
            pltpu.SemaphoreType.DMA,
        ],
        compiler_params=pltpu.CompilerParams(needs_layout_passes=False),
    )
    def sc_kernel(edges_hbm, out_hbm, s0, d0, s1, d1, acc0, acc1,
                  sem0, sem1, semo):
        wid = lax.axis_index("s") * 2 + lax.axis_index("c")
        g0 = wid
        g1 = wid + _NW

        c0s = pltpu.async_copy(edges_hbm.at[0, pl.ds(g0 * _EPG, _EPG)], s0, sem0)
        c0d = pltpu.async_copy(edges_hbm.at[1, pl.ds(g0 * _EPG, _EPG)], d0, sem0)

        @pl.when(g1 < _B)
        def _():
            pltpu.async_copy(edges_hbm.at[0, pl.ds(g1 * _EPG, _EPG)], s1, sem1)
            pltpu.async_copy(edges_hbm.at[1, pl.ds(g1 * _EPG, _EPG)], d1, sem1)

        _zero_acc(acc0)
        c0s.wait()
        c0d.wait()
        _histogram(s0, d0, acc0, g0 * _NPG)
        co0 = pltpu.async_copy(
            acc0, out_hbm.at[pl.ds(g0 * _NPG, _NPG), :], semo
        )

        @pl.when(g1 < _B)
        def _():
            _zero_acc(acc1)
            pltpu.make_async_copy(
                edges_hbm.at[0, pl.ds(g1 * _EPG, _EPG)], s1, sem1
            ).wait()
            pltpu.make_async_copy(
                edges_hbm.at[1, pl.ds(g1 * _EPG, _EPG)], d1, sem1
            ).wait()
            _histogram(s1, d1, acc1, g1 * _NPG)
            pltpu.sync_copy(acc1, out_hbm.at[pl.ds(g1 * _NPG, _NPG), :])

        co0.wait()

    return sc_kernel(edge_index)


_GPS = 10


def _tc_body(a_ref, h_ref, w_ref, b_ref, adj_ref, hb_ref, adjs_scr):
    s = pl.program_id(0)
    b = b_ref[...]
    w_bf = w_ref[...].astype(jnp.bfloat16)
    h_bf = h_ref[...].astype(jnp.bfloat16)
    hw_all = jax.lax.dot_general(
        h_bf, w_bf, (((1,), (0,)), ((), ())), preferred_element_type=jnp.float32
    )
    hwe_all = jnp.concatenate(
        [hw_all, jnp.ones((_GPS * _NPG, 1), jnp.float32)], axis=1
    ).astype(jnp.bfloat16)

    ik = lax.broadcasted_iota(jnp.int32, (_ADIM, _B * _ADIM), 0)
    jc = lax.broadcasted_iota(jnp.int32, (_ADIM, _B * _ADIM), 1)

    def gsl(i):
        return slice(i * _NPG, (i + 1) * _NPG)

    a_bf = [a_ref[gsl(i), :].astype(jnp.bfloat16) for i in range(_GPS)]

    agg_ext = [
        jax.lax.dot_general(
            a_bf[i], hwe_all[gsl(i), :], (((0,), (0,)), ((), ())),
            preferred_element_type=jnp.float32,
        )
        for i in range(_GPS)
    ]

    assigns = []
    for i in range(_GPS):
        aggw = agg_ext[i][:, :_ADIM]
        deg = agg_ext[i][:, _ADIM:]
        logits = aggw / jnp.maximum(deg, 1.0) + b
        m = jnp.max(logits, axis=1, keepdims=True)
        e = jnp.exp(logits - m)
        sm = e / jnp.sum(e, axis=1, keepdims=True)
        assigns.append(sm.astype(jnp.bfloat16))

    as_ = [
        jax.lax.dot_general(
            a_bf[i], assigns[i], (((1,), (0,)), ((), ())),
            preferred_element_type=jnp.float32,
        )
        for i in range(_GPS)
    ]

    pooled = [
        jax.lax.dot_general(
            assigns[i],
            jnp.concatenate(
                [h_bf[gsl(i), :], as_[i].astype(jnp.bfloat16)], axis=1
            ),
            (((0,), (0,)), ((), ())),
            preferred_element_type=jnp.float32,
        )
        for i in range(_GPS)
    ]

    for i in range(_GPS):
        hb_ref[i] = pooled[i][:, :_D]
        adjs_scr[pl.ds(s * _GPS + i, 1)] = pooled[i][jnp.newaxis, :, _D:]

    @pl.when(s == _B // _GPS - 1)
    def _():
        for g in range(_B):
            sel = (jc == ik + g * _ADIM).astype(jnp.bfloat16)
            adj_ref[g * _ADIM:(g + 1) * _ADIM, :] = jax.lax.dot_general(
                adjs_scr[g].astype(jnp.bfloat16), sel,
                (((1,), (0,)), ((), ())),
                preferred_element_type=jnp.float32,
            )


def _pool_tc(a, h, w_fc, b2d):
    return pl.pallas_call(
        _tc_body,
        grid=(_B // _GPS,),
        in_specs=[
            pl.BlockSpec((_GPS * _NPG, _NPG), lambda s: (s, 0)),
            pl.BlockSpec((_GPS * _NPG, _D), lambda s: (s, 0)),
            pl.BlockSpec((_D, _ADIM), lambda s: (0, 0)),
            pl.BlockSpec((1, _ADIM), lambda s: (0, 0)),
        ],
        out_specs=[
            pl.BlockSpec((_B * _ADIM, _B * _ADIM), lambda s: (0, 0)),
            pl.BlockSpec((_GPS, _ADIM, _D), lambda s: (s, 0, 0)),
        ],
        out_shape=[
            jax.ShapeDtypeStruct((_B * _ADIM, _B * _ADIM), jnp.float32),
            jax.ShapeDtypeStruct((_B, _ADIM, _D), jnp.float32),
        ],
        scratch_shapes=[pltpu.VMEM((_B, _ADIM, _ADIM), jnp.float32)],
        compiler_params=pltpu.CompilerParams(
            dimension_semantics=("arbitrary",),
        ),
    )(a, h, w_fc, b2d)


def kernel(h, edge_index, r, W_fc, b_fc):
    del r
    a = _build_adj_sc(edge_index)
    adj_new, h_batched = _pool_tc(a, h, W_fc, b_fc.reshape(1, _ADIM))
    return (adj_new, h_batched)


# speedup vs baseline: 113.0253x; 1.0123x over previous
import functools

import jax
import jax.numpy as jnp
from jax import lax
from jax.experimental import pallas as pl
from jax.experimental.pallas import tpu as pltpu
from jax.experimental.pallas import tpu_sc as plsc

_B = 50
_NPG = 200
_N = _B * _NPG
_EPG = 6400
_E = _B * _EPG
_D = 128
_ADIM = 25
_NW = 32
_GPW = 2


def _histogram(src_v, dst_v, acc_v, off):
    ones16 = jnp.ones((16,), jnp.float32)

    @plsc.parallel_loop(0, _EPG // 16, unroll=8)
    def _(i):
        s16 = src_v[pl.ds(i * 16, 16)]
        d16 = dst_v[pl.ds(i * 16, 16)]
        plsc.addupdate_scatter(acc_v, [s16 - off, d16 - off], ones16)


def _zero_acc(acc_v):
    z = jnp.zeros((16,), jnp.float32)

    @plsc.parallel_loop(0, _NPG, unroll=4)
    def _(r):
        for c in range(_NPG // 16):
            acc_v[r, pl.ds(c * 16, 16)] = z
        acc_v[r, pl.ds(_NPG - 16, 16)] = z


def _build_adj_sc(edge_index):
    mesh = plsc.VectorSubcoreMesh(core_axis_name="c", subcore_axis_name="s")

    @functools.partial(
        pl.kernel,
        out_type=jax.ShapeDtypeStruct((_N, _NPG), jnp.float32),
        mesh=mesh,
        scratch_types=[
            pltpu.VMEM((_EPG,), jnp.int32),
            pltpu.VMEM((_EPG,), jnp.int32),
            pltpu.VMEM((_EPG,), jnp.int32),
            pltpu.VMEM((_EPG,), jnp.int32),
            pltpu.VMEM((_NPG, _NPG), jnp.float32),
            pltpu.VMEM((_NPG, _NPG), jnp.float32),
            pltpu.SemaphoreType.DMA,
            pltpu.SemaphoreType.DMA,
            pltpu.SemaphoreType.DMA,
        ],
        compiler_params=pltpu.CompilerParams(needs_layout_passes=False),
    )
    def sc_kernel(edges_hbm, out_hbm, s0, d0, s1, d1, acc0, acc1,
                  sem0, sem1, semo):
        wid = lax.axis_index("s") * 2 + lax.axis_index("c")
        g0 = wid
        g1 = wid + _NW

        c0s = pltpu.async_copy(edges_hbm.at[0, pl.ds(g0 * _EPG, _EPG)], s0, sem0)
        c0d = pltpu.async_copy(edges_hbm.at[1, pl.ds(g0 * _EPG, _EPG)], d0, sem0)

        @pl.when(g1 < _B)
        def _():
            pltpu.async_copy(edges_hbm.at[0, pl.ds(g1 * _EPG, _EPG)], s1, sem1)
            pltpu.async_copy(edges_hbm.at[1, pl.ds(g1 * _EPG, _EPG)], d1, sem1)

        _zero_acc(acc0)
        c0s.wait()
        c0d.wait()
        _histogram(s0, d0, acc0, g0 * _NPG)
        co0 = pltpu.async_copy(
            acc0, out_hbm.at[pl.ds(g0 * _NPG, _NPG), :], semo
        )

        @pl.when(g1 < _B)
        def _():
            _zero_acc(acc1)
            pltpu.make_async_copy(
                edges_hbm.at[0, pl.ds(g1 * _EPG, _EPG)], s1, sem1
            ).wait()
            pltpu.make_async_copy(
                edges_hbm.at[1, pl.ds(g1 * _EPG, _EPG)], d1, sem1
            ).wait()
            _histogram(s1, d1, acc1, g1 * _NPG)
            pltpu.sync_copy(acc1, out_hbm.at[pl.ds(g1 * _NPG, _NPG), :])

        co0.wait()

    return sc_kernel(edge_index)


_GPS = 25


def _tc_body(a_ref, h_ref, w_ref, b_ref, adj_ref, hb_ref, adjs_scr):
    s = pl.program_id(0)
    b = b_ref[...]
    w_bf = w_ref[...].astype(jnp.bfloat16)
    h_bf = h_ref[...].astype(jnp.bfloat16)
    hw_all = jax.lax.dot_general(
        h_bf, w_bf, (((1,), (0,)), ((), ())), preferred_element_type=jnp.float32
    )
    hwe_all = jnp.concatenate(
        [hw_all, jnp.ones((_GPS * _NPG, 1), jnp.float32)], axis=1
    ).astype(jnp.bfloat16)

    ik = lax.broadcasted_iota(jnp.int32, (_ADIM, _B * _ADIM), 0)
    jc = lax.broadcasted_iota(jnp.int32, (_ADIM, _B * _ADIM), 1)

    def gsl(i):
        return slice(i * _NPG, (i + 1) * _NPG)

    a_bf = [a_ref[gsl(i), :].astype(jnp.bfloat16) for i in range(_GPS)]

    agg_ext = [
        jax.lax.dot_general(
            a_bf[i], hwe_all[gsl(i), :], (((0,), (0,)), ((), ())),
            preferred_element_type=jnp.float32,
        )
        for i in range(_GPS)
    ]

    assigns = []
    for i in range(_GPS):
        aggw = agg_ext[i][:, :_ADIM]
        deg = agg_ext[i][:, _ADIM:]
        logits = aggw / jnp.maximum(deg, 1.0) + b
        m = jnp.max(logits, axis=1, keepdims=True)
        e = jnp.exp(logits - m)
        sm = e / jnp.sum(e, axis=1, keepdims=True)
        assigns.append(sm.astype(jnp.bfloat16))

    as_ = [
        jax.lax.dot_general(
            a_bf[i], assigns[i], (((1,), (0,)), ((), ())),
            preferred_element_type=jnp.float32,
        )
        for i in range(_GPS)
    ]

    pooled = [
        jax.lax.dot_general(
            assigns[i],
            jnp.concatenate(
                [h_bf[gsl(i), :], as_[i].astype(jnp.bfloat16)], axis=1
            ),
            (((0,), (0,)), ((), ())),
            preferred_element_type=jnp.float32,
        )
        for i in range(_GPS)
    ]

    for i in range(_GPS):
        hb_ref[i] = pooled[i][:, :_D]
        adjs_scr[pl.ds(s * _GPS + i, 1)] = pooled[i][jnp.newaxis, :, _D:]

    @pl.when(s == _B // _GPS - 1)
    def _():
        for g in range(_B):
            sel = (jc == ik + g * _ADIM).astype(jnp.bfloat16)
            adj_ref[g * _ADIM:(g + 1) * _ADIM, :] = jax.lax.dot_general(
                adjs_scr[g].astype(jnp.bfloat16), sel,
                (((1,), (0,)), ((), ())),
                preferred_element_type=jnp.float32,
            )


def _pool_tc(a, h, w_fc, b2d):
    return pl.pallas_call(
        _tc_body,
        grid=(_B // _GPS,),
        in_specs=[
            pl.BlockSpec((_GPS * _NPG, _NPG), lambda s: (s, 0)),
            pl.BlockSpec((_GPS * _NPG, _D), lambda s: (s, 0)),
            pl.BlockSpec((_D, _ADIM), lambda s: (0, 0)),
            pl.BlockSpec((1, _ADIM), lambda s: (0, 0)),
        ],
        out_specs=[
            pl.BlockSpec((_B * _ADIM, _B * _ADIM), lambda s: (0, 0)),
            pl.BlockSpec((_GPS, _ADIM, _D), lambda s: (s, 0, 0)),
        ],
        out_shape=[
            jax.ShapeDtypeStruct((_B * _ADIM, _B * _ADIM), jnp.float32),
            jax.ShapeDtypeStruct((_B, _ADIM, _D), jnp.float32),
        ],
        scratch_shapes=[pltpu.VMEM((_B, _ADIM, _ADIM), jnp.float32)],
        compiler_params=pltpu.CompilerParams(
            dimension_semantics=("arbitrary",),
        ),
    )(a, h, w_fc, b2d)


def kernel(h, edge_index, r, W_fc, b_fc):
    del r
    a = _build_adj_sc(edge_index)
    adj_new, h_batched = _pool_tc(a, h, W_fc, b_fc.reshape(1, _ADIM))
    return (adj_new, h_batched)


# speedup vs baseline: 117.8108x; 1.0423x over previous
import functools

import jax
import jax.numpy as jnp
from jax import lax
from jax.experimental import pallas as pl
from jax.experimental.pallas import tpu as pltpu
from jax.experimental.pallas import tpu_sc as plsc

_B = 50
_NPG = 200
_N = _B * _NPG
_EPG = 6400
_E = _B * _EPG
_D = 128
_ADIM = 25
_NW = 32
_GPW = 2


def _histogram(src_v, dst_v, acc_v, off):
    ones16 = jnp.ones((16,), jnp.float32)

    @plsc.parallel_loop(0, _EPG // 16, unroll=8)
    def _(i):
        s16 = src_v[pl.ds(i * 16, 16)]
        d16 = dst_v[pl.ds(i * 16, 16)]
        plsc.addupdate_scatter(acc_v, [s16 - off, d16 - off], ones16)


def _zero_acc(acc_v):
    z = jnp.zeros((16,), jnp.float32)

    @plsc.parallel_loop(0, _NPG, unroll=4)
    def _(r):
        for c in range(_NPG // 16):
            acc_v[r, pl.ds(c * 16, 16)] = z
        acc_v[r, pl.ds(_NPG - 16, 16)] = z


def _build_adj_sc(edge_index):
    mesh = plsc.VectorSubcoreMesh(core_axis_name="c", subcore_axis_name="s")

    @functools.partial(
        pl.kernel,
        out_type=jax.ShapeDtypeStruct((_N, _NPG), jnp.float32),
        mesh=mesh,
        scratch_types=[
            pltpu.VMEM((_EPG,), jnp.int32),
            pltpu.VMEM((_EPG,), jnp.int32),
            pltpu.VMEM((_EPG,), jnp.int32),
            pltpu.VMEM((_EPG,), jnp.int32),
            pltpu.VMEM((_NPG, _NPG), jnp.float32),
            pltpu.VMEM((_NPG, _NPG), jnp.float32),
            pltpu.SemaphoreType.DMA,
            pltpu.SemaphoreType.DMA,
            pltpu.SemaphoreType.DMA,
        ],
        compiler_params=pltpu.CompilerParams(needs_layout_passes=False),
    )
    def sc_kernel(edges_hbm, out_hbm, s0, d0, s1, d1, acc0, acc1,
                  sem0, sem1, semo):
        wid = lax.axis_index("s") * 2 + lax.axis_index("c")
        g0 = wid
        g1 = wid + _NW

        c0s = pltpu.async_copy(edges_hbm.at[0, pl.ds(g0 * _EPG, _EPG)], s0, sem0)
        c0d = pltpu.async_copy(edges_hbm.at[1, pl.ds(g0 * _EPG, _EPG)], d0, sem0)

        @pl.when(g1 < _B)
        def _():
            pltpu.async_copy(edges_hbm.at[0, pl.ds(g1 * _EPG, _EPG)], s1, sem1)
            pltpu.async_copy(edges_hbm.at[1, pl.ds(g1 * _EPG, _EPG)], d1, sem1)

        _zero_acc(acc0)
        c0s.wait()
        c0d.wait()
        _histogram(s0, d0, acc0, g0 * _NPG)
        co0 = pltpu.async_copy(
            acc0, out_hbm.at[pl.ds(g0 * _NPG, _NPG), :], semo
        )

        @pl.when(g1 < _B)
        def _():
            _zero_acc(acc1)
            pltpu.make_async_copy(
                edges_hbm.at[0, pl.ds(g1 * _EPG, _EPG)], s1, sem1
            ).wait()
            pltpu.make_async_copy(
                edges_hbm.at[1, pl.ds(g1 * _EPG, _EPG)], d1, sem1
            ).wait()
            _histogram(s1, d1, acc1, g1 * _NPG)
            pltpu.sync_copy(acc1, out_hbm.at[pl.ds(g1 * _NPG, _NPG), :])

        co0.wait()

    return sc_kernel(edge_index)


_GPS = 25


def _tc_body(a_ref, h_ref, w_ref, b_ref, adj_ref, hb_ref, adjs_scr):
    s = pl.program_id(0)
    b = b_ref[...]
    w_bf = w_ref[...].astype(jnp.bfloat16)
    h_bf = h_ref[...].astype(jnp.bfloat16)
    hw_all = jax.lax.dot_general(
        h_bf, w_bf, (((1,), (0,)), ((), ())), preferred_element_type=jnp.float32
    )
    hwe_all = jnp.concatenate(
        [hw_all, jnp.ones((_GPS * _NPG, 1), jnp.float32)], axis=1
    ).astype(jnp.bfloat16)

    ik = lax.broadcasted_iota(jnp.int32, (_ADIM, _B * _ADIM), 0)
    jc = lax.broadcasted_iota(jnp.int32, (_ADIM, _B * _ADIM), 1)

    def gsl(i):
        return slice(i * _NPG, (i + 1) * _NPG)

    a_bf = [a_ref[gsl(i), :].astype(jnp.bfloat16) for i in range(_GPS)]

    agg_ext = [
        jax.lax.dot_general(
            a_bf[i], hwe_all[gsl(i), :], (((0,), (0,)), ((), ())),
            preferred_element_type=jnp.float32,
        )
        for i in range(_GPS)
    ]

    assigns = []
    for i in range(_GPS):
        aggw = agg_ext[i][:, :_ADIM]
        deg = agg_ext[i][:, _ADIM:]
        logits = aggw / jnp.maximum(deg, 1.0) + b
        m = jnp.max(logits, axis=1, keepdims=True)
        e = jnp.exp(logits - m)
        sm = e / jnp.sum(e, axis=1, keepdims=True)
        assigns.append(sm.astype(jnp.bfloat16))

    as_ = [
        jax.lax.dot_general(
            a_bf[i], assigns[i], (((1,), (0,)), ((), ())),
            preferred_element_type=jnp.float32,
        )
        for i in range(_GPS)
    ]

    pooled = [
        jax.lax.dot_general(
            assigns[i],
            jnp.concatenate(
                [h_bf[gsl(i), :], as_[i].astype(jnp.bfloat16)], axis=1
            ),
            (((0,), (0,)), ((), ())),
            preferred_element_type=jnp.float32,
        )
        for i in range(_GPS)
    ]

    for i in range(_GPS):
        adjs_scr[pl.ds(s * _GPS + i, 1)] = pooled[i][jnp.newaxis, :, :]

    @pl.when(s == _B // _GPS - 1)
    def _():
        for g in range(_B):
            p = adjs_scr[g]
            hb_ref[:, g:g + 1, :] = p[:, :_D].reshape(_ADIM, 1, _D)
            sel = (jc == ik + g * _ADIM).astype(jnp.bfloat16)
            adj_ref[g * _ADIM:(g + 1) * _ADIM, :] = jax.lax.dot_general(
                p[:, _D:].astype(jnp.bfloat16), sel,
                (((1,), (0,)), ((), ())),
                preferred_element_type=jnp.float32,
            )


def _pool_tc(a, h, w_fc, b2d):
    return pl.pallas_call(
        _tc_body,
        grid=(_B // _GPS,),
        in_specs=[
            pl.BlockSpec((_GPS * _NPG, _NPG), lambda s: (s, 0)),
            pl.BlockSpec((_GPS * _NPG, _D), lambda s: (s, 0)),
            pl.BlockSpec((_D, _ADIM), lambda s: (0, 0)),
            pl.BlockSpec((1, _ADIM), lambda s: (0, 0)),
        ],
        out_specs=[
            pl.BlockSpec((_B * _ADIM, _B * _ADIM), lambda s: (0, 0)),
            pl.BlockSpec((_ADIM, _B, _D), lambda s: (0, 0, 0)),
        ],
        out_shape=[
            jax.ShapeDtypeStruct((_B * _ADIM, _B * _ADIM), jnp.float32),
            jax.ShapeDtypeStruct((_ADIM, _B, _D), jnp.float32),
        ],
        scratch_shapes=[pltpu.VMEM((_B, _ADIM, _D + _ADIM), jnp.float32)],
        compiler_params=pltpu.CompilerParams(
            dimension_semantics=("arbitrary",),
        ),
    )(a, h, w_fc, b2d)


def kernel(h, edge_index, r, W_fc, b_fc):
    del r
    a = _build_adj_sc(edge_index)
    adj_new, hb_t = _pool_tc(a, h, W_fc, b_fc.reshape(1, _ADIM))
    return (adj_new, jnp.swapaxes(hb_t, 0, 1))
